# baseline (speedup 1.0000x reference)
import functools

import jax
import jax.numpy as jnp
from jax.experimental import pallas as pl
from jax.experimental.pallas import tpu as pltpu

_T = 256
_C = 128
_L = 128
_F = 6
_ENC_DIL = (1, 2, 4, 8)
_DEC_DIL = (8, 4, 2, 1, 1)
_ENC_LN = (True, True, True, True)
_DEC_LN = (True, True, True, True, False)
_LN_EPS = 1e-5
_SLOPE = 0.1
_MAXD = 8


def _conv_stack_body(x_ref, w_ref, b_ref, g_ref, beta_ref, o_ref, buf,
                     *, bh, cin, dilations, apply_ln, out_f32_slice):
    nl = len(dilations)
    if cin < _C:
        buf[...] = jnp.zeros(buf.shape, jnp.bfloat16)
        buf[:, _MAXD:_MAXD + _T, 0:cin] = x_ref[...].astype(jnp.bfloat16)
    else:
        buf[:, 0:_MAXD, :] = jnp.zeros((bh, _MAXD, _C), jnp.bfloat16)
        buf[:, _MAXD + _T:, :] = jnp.zeros((bh, _MAXD, _C), jnp.bfloat16)
        buf[:, _MAXD:_MAXD + _T, :] = x_ref[...].astype(jnp.bfloat16)

    ln_idx = 0
    for l in range(nl):
        d = dilations[l]
        y = None
        for tap in range(3):
            base = _MAXD + (tap - 1) * d
            lhs = buf[:, base:base + _T, :].reshape(bh * _T, _C)
            p = jnp.dot(lhs, w_ref[l, tap * _C:(tap + 1) * _C, :],
                        preferred_element_type=jnp.float32)
            y = p if y is None else y + p
        y = y.reshape(bh, _T, _C) + b_ref[l]
        if apply_ln[l]:
            y = jnp.where(y >= 0, y, _SLOPE * y)
            mu = jnp.mean(y, axis=(1, 2), keepdims=True)
            msq = jnp.mean(y * y, axis=(1, 2), keepdims=True)
            var = jnp.maximum(msq - mu * mu, 0.0)
            y = (y - mu) * jax.lax.rsqrt(var + _LN_EPS)
            y = y * g_ref[ln_idx] + beta_ref[ln_idx]
            ln_idx += 1
        if l + 1 < nl:
            buf[:, _MAXD:_MAXD + _T, :] = y.astype(jnp.bfloat16)
        elif out_f32_slice:
            o_ref[...] = y[:, :, 0:_F]
        else:
            o_ref[...] = y.astype(jnp.bfloat16)


def _run_conv_stack(x, w, b, g, beta, *, dilations, apply_ln, out_f32_slice):
    B = x.shape[0]
    cin = x.shape[2]
    nb = 2 if B % 2 == 0 else 1
    bh = B // nb
    nl, threec, _ = w.shape
    nln = g.shape[0]
    if out_f32_slice:
        out_shape = jax.ShapeDtypeStruct((B, _T, _F), jnp.float32)
        out_spec = pl.BlockSpec((bh, _T, _F), lambda i: (i, 0, 0))
    else:
        out_shape = jax.ShapeDtypeStruct((B, _T, _C), jnp.bfloat16)
        out_spec = pl.BlockSpec((bh, _T, _C), lambda i: (i, 0, 0))
    body = functools.partial(
        _conv_stack_body, bh=bh, cin=cin, dilations=dilations,
        apply_ln=apply_ln, out_f32_slice=out_f32_slice)
    return pl.pallas_call(
        body,
        out_shape=out_shape,
        grid=(nb,),
        in_specs=[
            pl.BlockSpec((bh, _T, cin), lambda i: (i, 0, 0)),
            pl.BlockSpec((nl, threec, _C), lambda i: (0, 0, 0)),
            pl.BlockSpec((nl, 1, _C), lambda i: (0, 0, 0)),
            pl.BlockSpec((nln, _T, _C), lambda i: (0, 0, 0)),
            pl.BlockSpec((nln, _T, _C), lambda i: (0, 0, 0)),
        ],
        out_specs=out_spec,
        scratch_shapes=[pltpu.VMEM((bh, _T + 2 * _MAXD, _C), jnp.bfloat16)],
        compiler_params=pltpu.CompilerParams(
            dimension_semantics=("parallel",)),
    )(x, w, b, g, beta)


def _fc_body(flat_ref, w_ref, b_ref, y_ref):
    y_ref[...] = jnp.dot(flat_ref[...], w_ref[...],
                         preferred_element_type=jnp.float32) + b_ref[...]


def _dec_in_body(y_ref, eps_ref, w_ref, b_ref, h_ref):
    mean = y_ref[:, 0:_L]
    logvar = y_ref[:, _L:]
    z = (mean + eps_ref[...] * jnp.exp(0.5 * logvar)).astype(jnp.bfloat16)
    ncols = h_ref.shape[1]
    step = 2048
    for j in range(0, ncols, step):
        h_ref[:, j:j + step] = (
            jnp.dot(z, w_ref[:, j:j + step],
                    preferred_element_type=jnp.float32)
            + b_ref[:, j:j + step]).astype(jnp.bfloat16)


def kernel(x, eps, enc_w, enc_b, enc_g, enc_beta, fc_w, fc_b,
           dec_in_w, dec_in_b, dec_w, dec_b, dec_g, dec_beta):
    B = x.shape[0]
    flat_dim = _T * _C

    enc = _run_conv_stack(x, enc_w, enc_b, enc_g, enc_beta,
                          dilations=_ENC_DIL, apply_ln=_ENC_LN,
                          out_f32_slice=False)
    flat = enc.reshape(B, flat_dim)

    y = pl.pallas_call(
        _fc_body,
        out_shape=jax.ShapeDtypeStruct((B, 2 * _L), jnp.float32),
        grid=(2,),
        in_specs=[
            pl.BlockSpec((B, flat_dim), lambda i: (0, 0)),
            pl.BlockSpec((flat_dim, _L), lambda i: (0, i)),
            pl.BlockSpec((1, _L), lambda i: (0, i)),
        ],
        out_specs=pl.BlockSpec((B, _L), lambda i: (0, i)),
        compiler_params=pltpu.CompilerParams(
            dimension_semantics=("parallel",)),
    )(flat, fc_w, fc_b)
    mean = y[:, 0:_L]
    logvar = y[:, _L:]

    h = pl.pallas_call(
        _dec_in_body,
        out_shape=jax.ShapeDtypeStruct((B, flat_dim), jnp.bfloat16),
        grid=(2,),
        in_specs=[
            pl.BlockSpec((B, 2 * _L), lambda i: (0, 0)),
            pl.BlockSpec((B, _L), lambda i: (0, 0)),
            pl.BlockSpec((_L, flat_dim // 2), lambda i: (0, i)),
            pl.BlockSpec((1, flat_dim // 2), lambda i: (0, i)),
        ],
        out_specs=pl.BlockSpec((B, flat_dim // 2), lambda i: (0, i)),
        compiler_params=pltpu.CompilerParams(
            dimension_semantics=("parallel",)),
    )(y, eps, dec_in_w, dec_in_b)
    hc = h.reshape(B, _T, _C)

    recon = _run_conv_stack(hc, dec_w, dec_b, dec_g, dec_beta,
                            dilations=_DEC_DIL, apply_ln=_DEC_LN,
                            out_f32_slice=True)
    return recon, mean, logvar


# speedup vs baseline: 1.0439x; 1.0439x over previous
import functools

import jax
import jax.numpy as jnp
from jax.experimental import pallas as pl
from jax.experimental.pallas import tpu as pltpu

_T = 256
_C = 128
_L = 128
_F = 6
_ENC_DIL = (1, 2, 4, 8)
_DEC_DIL = (8, 4, 2, 1, 1)
_ENC_LN = (True, True, True, True)
_DEC_LN = (True, True, True, True, False)
_LN_EPS = 1e-5
_SLOPE = 0.1
_MAXD = 8
_FLAT = _T * _C


def _conv_layers(buf, w_ref, b_ref, g_ref, beta_ref, bh, dilations, apply_ln):
    nl = len(dilations)
    ln_idx = 0
    for l in range(nl):
        d = dilations[l]
        y = None
        for tap in range(3):
            base = _MAXD + (tap - 1) * d
            lhs = buf[:, base:base + _T, :].reshape(bh * _T, _C)
            p = jnp.dot(lhs, w_ref[l, tap * _C:(tap + 1) * _C, :],
                        preferred_element_type=jnp.float32)
            y = p if y is None else y + p
        y = y.reshape(bh, _T, _C) + b_ref[l]
        if apply_ln[l]:
            y = jnp.where(y >= 0, y, _SLOPE * y)
            mu = jnp.mean(y, axis=(1, 2), keepdims=True)
            msq = jnp.mean(y * y, axis=(1, 2), keepdims=True)
            var = jnp.maximum(msq - mu * mu, 0.0)
            y = (y - mu) * jax.lax.rsqrt(var + _LN_EPS)
            y = y * g_ref[ln_idx] + beta_ref[ln_idx]
            ln_idx += 1
        if l + 1 < nl:
            buf[:, _MAXD:_MAXD + _T, :] = y.astype(jnp.bfloat16)
    return y


def _mega_body(x_ref, eps_ref, enc_w_ref, enc_b_ref, enc_g_ref, enc_beta_ref,
               fc_w_ref, fc_b_ref, dec_in_w_ref, dec_in_b_ref,
               dec_w_ref, dec_b_ref, dec_g_ref, dec_beta_ref,
               recon_ref, mean_ref, logvar_ref, buf, *, bh):
    buf[...] = jnp.zeros(buf.shape, jnp.bfloat16)
    buf[:, _MAXD:_MAXD + _T, 0:_F] = x_ref[...].astype(jnp.bfloat16)
    y = _conv_layers(buf, enc_w_ref, enc_b_ref, enc_g_ref, enc_beta_ref,
                     bh, _ENC_DIL, _ENC_LN)

    flat = y.astype(jnp.bfloat16).reshape(bh, _FLAT)
    y2 = jnp.dot(flat, fc_w_ref[...],
                 preferred_element_type=jnp.float32) + fc_b_ref[...]
    mean = y2[:, 0:_L]
    logvar = y2[:, _L:]
    mean_ref[...] = mean
    logvar_ref[...] = logvar

    z = (mean + eps_ref[...] * jnp.exp(0.5 * logvar)).astype(jnp.bfloat16)
    step = 2048
    ts = step // _C
    for j in range(0, _FLAT, step):
        hj = jnp.dot(z, dec_in_w_ref[:, j:j + step],
                     preferred_element_type=jnp.float32) + \
            dec_in_b_ref[:, j:j + step]
        t0 = _MAXD + j // _C
        buf[:, t0:t0 + ts, :] = hj.astype(jnp.bfloat16).reshape(bh, ts, _C)

    y = _conv_layers(buf, dec_w_ref, dec_b_ref, dec_g_ref, dec_beta_ref,
                     bh, _DEC_DIL, _DEC_LN)
    recon_ref[...] = y[:, :, 0:_F]


def kernel(x, eps, enc_w, enc_b, enc_g, enc_beta, fc_w, fc_b,
           dec_in_w, dec_in_b, dec_w, dec_b, dec_g, dec_beta):
    B = x.shape[0]
    nb = 2 if B % 2 == 0 else 1
    bh = B // nb
    body = functools.partial(_mega_body, bh=bh)
    recon, mean, logvar = pl.pallas_call(
        body,
        out_shape=(jax.ShapeDtypeStruct((B, _T, _F), jnp.float32),
                   jax.ShapeDtypeStruct((B, _L), jnp.float32),
                   jax.ShapeDtypeStruct((B, _L), jnp.float32)),
        grid=(nb,),
        in_specs=[
            pl.BlockSpec((bh, _T, _F), lambda i: (i, 0, 0)),
            pl.BlockSpec((bh, _L), lambda i: (i, 0)),
            pl.BlockSpec(enc_w.shape, lambda i: (0, 0, 0)),
            pl.BlockSpec(enc_b.shape, lambda i: (0, 0, 0)),
            pl.BlockSpec(enc_g.shape, lambda i: (0, 0, 0)),
            pl.BlockSpec(enc_beta.shape, lambda i: (0, 0, 0)),
            pl.BlockSpec(fc_w.shape, lambda i: (0, 0)),
            pl.BlockSpec(fc_b.shape, lambda i: (0, 0)),
            pl.BlockSpec(dec_in_w.shape, lambda i: (0, 0)),
            pl.BlockSpec(dec_in_b.shape, lambda i: (0, 0)),
            pl.BlockSpec(dec_w.shape, lambda i: (0, 0, 0)),
            pl.BlockSpec(dec_b.shape, lambda i: (0, 0, 0)),
            pl.BlockSpec(dec_g.shape, lambda i: (0, 0, 0)),
            pl.BlockSpec(dec_beta.shape, lambda i: (0, 0, 0)),
        ],
        out_specs=(
            pl.BlockSpec((bh, _T, _F), lambda i: (i, 0, 0)),
            pl.BlockSpec((bh, _L), lambda i: (i, 0)),
            pl.BlockSpec((bh, _L), lambda i: (i, 0)),
        ),
        scratch_shapes=[pltpu.VMEM((bh, _T + 2 * _MAXD, _C), jnp.bfloat16)],
        compiler_params=pltpu.CompilerParams(
            dimension_semantics=("parallel",)),
    )(x, eps, enc_w, enc_b, enc_g, enc_beta, fc_w, fc_b,
      dec_in_w, dec_in_b, dec_w, dec_b, dec_g, dec_beta)
    return recon, mean, logvar


# speedup vs baseline: 1.4280x; 1.3680x over previous
import functools

import jax
import jax.numpy as jnp
from jax.experimental import pallas as pl
from jax.experimental.pallas import tpu as pltpu

_T = 256
_C = 128
_L = 128
_F = 6
_ENC_DIL = (1, 2, 4, 8)
_DEC_DIL = (8, 4, 2, 1, 1)
_ENC_LN = (True, True, True, True)
_DEC_LN = (True, True, True, True, False)
_LN_EPS = 1e-5
_SLOPE = 0.1
_MAXD = 8
_FLAT = _T * _C


def _conv_layers(buf, w_ref, b_ref, g_ref, beta_ref, bh, dilations, apply_ln):
    nl = len(dilations)
    ln_idx = 0
    for l in range(nl):
        d = dilations[l]
        y = None
        for tap in range(3):
            base = _MAXD + (tap - 1) * d
            lhs = buf[:, base:base + _T, :].reshape(bh * _T, _C)
            p = jnp.dot(lhs, w_ref[l, tap * _C:(tap + 1) * _C, :],
                        preferred_element_type=jnp.float32)
            y = p if y is None else y + p
        y = y.reshape(bh, _T, _C) + b_ref[l]
        if apply_ln[l]:
            y = jnp.where(y >= 0, y, _SLOPE * y)
            mu = jnp.mean(y, axis=(1, 2), keepdims=True)
            msq = jnp.mean(y * y, axis=(1, 2), keepdims=True)
            var = jnp.maximum(msq - mu * mu, 0.0)
            y = (y - mu) * jax.lax.rsqrt(var + _LN_EPS)
            y = y * g_ref[ln_idx] + beta_ref[ln_idx]
            ln_idx += 1
        if l + 1 < nl:
            buf[:, _MAXD:_MAXD + _T, :] = y.astype(jnp.bfloat16)
    return y


def _mega_body(x_ref, eps_ref, enc_w_ref, enc_b_ref, enc_g_ref, enc_beta_ref,
               fc_w_hbm, fc_b_ref, dec_in_w_hbm, dec_in_b_ref,
               dec_w_hbm, dec_b_ref, dec_g_hbm, dec_beta_hbm,
               recon_ref, mean_ref, logvar_ref,
               buf, fc_w_v, dec_in_w_v, dec_w_v, dec_g_v, dec_beta_v, sems,
               *, bh):
    cp_fc = pltpu.make_async_copy(fc_w_hbm, fc_w_v, sems.at[0])
    cp_di = pltpu.make_async_copy(dec_in_w_hbm, dec_in_w_v, sems.at[1])
    cp_dw = pltpu.make_async_copy(dec_w_hbm, dec_w_v, sems.at[2])
    cp_dg = pltpu.make_async_copy(dec_g_hbm, dec_g_v, sems.at[3])
    cp_db = pltpu.make_async_copy(dec_beta_hbm, dec_beta_v, sems.at[4])
    cp_fc.start()
    cp_di.start()
    cp_dw.start()
    cp_dg.start()
    cp_db.start()

    buf[...] = jnp.zeros(buf.shape, jnp.bfloat16)
    buf[:, _MAXD:_MAXD + _T, 0:_F] = x_ref[...].astype(jnp.bfloat16)
    y = _conv_layers(buf, enc_w_ref, enc_b_ref, enc_g_ref, enc_beta_ref,
                     bh, _ENC_DIL, _ENC_LN)

    flat = y.astype(jnp.bfloat16).reshape(bh, _FLAT)
    cp_fc.wait()
    y2 = jnp.dot(flat, fc_w_v[...],
                 preferred_element_type=jnp.float32) + fc_b_ref[...]
    mean = y2[:, 0:_L]
    logvar = y2[:, _L:]
    mean_ref[...] = mean
    logvar_ref[...] = logvar

    z = (mean + eps_ref[...] * jnp.exp(0.5 * logvar)).astype(jnp.bfloat16)
    cp_di.wait()
    step = 2048
    ts = step // _C
    for j in range(0, _FLAT, step):
        hj = jnp.dot(z, dec_in_w_v[:, j:j + step],
                     preferred_element_type=jnp.float32) + \
            dec_in_b_ref[:, j:j + step]
        t0 = _MAXD + j // _C
        buf[:, t0:t0 + ts, :] = hj.astype(jnp.bfloat16).reshape(bh, ts, _C)

    cp_dw.wait()
    cp_dg.wait()
    cp_db.wait()
    y = _conv_layers(buf, dec_w_v, dec_b_ref, dec_g_v, dec_beta_v,
                     bh, _DEC_DIL, _DEC_LN)
    recon_ref[...] = y[:, :, 0:_F]


def kernel(x, eps, enc_w, enc_b, enc_g, enc_beta, fc_w, fc_b,
           dec_in_w, dec_in_b, dec_w, dec_b, dec_g, dec_beta):
    B = x.shape[0]
    body = functools.partial(_mega_body, bh=B)
    any_spec = pl.BlockSpec(memory_space=pl.ANY)
    recon, mean, logvar = pl.pallas_call(
        body,
        out_shape=(jax.ShapeDtypeStruct((B, _T, _F), jnp.float32),
                   jax.ShapeDtypeStruct((B, _L), jnp.float32),
                   jax.ShapeDtypeStruct((B, _L), jnp.float32)),
        in_specs=[
            pl.BlockSpec(x.shape, lambda: (0, 0, 0)),
            pl.BlockSpec(eps.shape, lambda: (0, 0)),
            pl.BlockSpec(enc_w.shape, lambda: (0, 0, 0)),
            pl.BlockSpec(enc_b.shape, lambda: (0, 0, 0)),
            pl.BlockSpec(enc_g.shape, lambda: (0, 0, 0)),
            pl.BlockSpec(enc_beta.shape, lambda: (0, 0, 0)),
            any_spec,
            pl.BlockSpec(fc_b.shape, lambda: (0, 0)),
            any_spec,
            pl.BlockSpec(dec_in_b.shape, lambda: (0, 0)),
            any_spec,
            pl.BlockSpec(dec_b.shape, lambda: (0, 0, 0)),
            any_spec,
            any_spec,
        ],
        out_specs=(
            pl.BlockSpec((B, _T, _F), lambda: (0, 0, 0)),
            pl.BlockSpec((B, _L), lambda: (0, 0)),
            pl.BlockSpec((B, _L), lambda: (0, 0)),
        ),
        scratch_shapes=[
            pltpu.VMEM((B, _T + 2 * _MAXD, _C), jnp.bfloat16),
            pltpu.VMEM(fc_w.shape, fc_w.dtype),
            pltpu.VMEM(dec_in_w.shape, dec_in_w.dtype),
            pltpu.VMEM(dec_w.shape, dec_w.dtype),
            pltpu.VMEM(dec_g.shape, dec_g.dtype),
            pltpu.VMEM(dec_beta.shape, dec_beta.dtype),
            pltpu.SemaphoreType.DMA((5,)),
        ],
    )(x, eps, enc_w, enc_b, enc_g, enc_beta, fc_w, fc_b,
      dec_in_w, dec_in_b, dec_w, dec_b, dec_g, dec_beta)
    return recon, mean, logvar
